# baseline (speedup 1.0000x reference)
import functools

import jax
import jax.numpy as jnp
from jax import lax
from jax.experimental import pallas as pl
from jax.experimental.pallas import tpu as pltpu
from jax.experimental.pallas import tpu_sc as plsc

NC = 2
NS = 16
NW = NC * NS

CHUNK = 1600
NBUF = 2


def _gather_body(n_per_w, n_chunks, idx_hbm, table_hbm, out_hbm,
                 idx_v, rows_v, sem_i0, sem_i1, sem_g, sem_s0, sem_s1):
    wid = lax.axis_index("s") * NC + lax.axis_index("c")
    base = wid * n_per_w
    sem_i = (sem_i0, sem_i1)
    sem_s = (sem_s0, sem_s1)

    def start_idx(j, p):
        return pltpu.async_copy(
            idx_hbm.at[pl.ds(base + j * CHUNK, CHUNK)], idx_v.at[p], sem_i[p])

    idx_pending = {0: start_idx(0, 0), 1: start_idx(1, 1)}
    store_pending = {}

    for j in range(n_chunks):
        p = j % NBUF
        if j >= NBUF:
            store_pending.pop(j - NBUF).wait()
        idx_pending.pop(j).wait()
        gather = pltpu.async_copy(table_hbm.at[idx_v.at[p]], rows_v.at[p],
                                  sem_g)
        if j + NBUF < n_chunks:
            pass
        gather.wait()
        if j + NBUF < n_chunks:
            idx_pending[j + NBUF] = start_idx(j + NBUF, p)
        store_pending[j] = pltpu.async_copy(
            rows_v.at[p], out_hbm.at[pl.ds(base + j * CHUNK, CHUNK)], sem_s[p])

    for j in sorted(store_pending):
        store_pending.pop(j).wait()


def kernel(input, weight):
    B0, B1 = input.shape
    V, D = weight.shape
    N = B0 * B1
    assert N % (NW * CHUNK) == 0
    n_per_w = N // NW
    n_chunks = n_per_w // CHUNK

    idx = input.reshape(N).astype(jnp.int32)

    mesh = plsc.VectorSubcoreMesh(core_axis_name="c", subcore_axis_name="s")
    run = pl.kernel(
        functools.partial(_gather_body, n_per_w, n_chunks),
        out_type=jax.ShapeDtypeStruct((N, D), jnp.float32),
        mesh=mesh,
        scratch_types=[
            pltpu.VMEM((NBUF, CHUNK), jnp.int32),
            pltpu.VMEM((NBUF, CHUNK, D), jnp.float32),
            pltpu.SemaphoreType.DMA,
            pltpu.SemaphoreType.DMA,
            pltpu.SemaphoreType.DMA,
            pltpu.SemaphoreType.DMA,
            pltpu.SemaphoreType.DMA,
        ],
        compiler_params=pltpu.CompilerParams(use_tc_tiling_on_sc=False),
    )
    out = run(idx, weight)
    return out.reshape(B0, B1, D)


# speedup vs baseline: 1.0005x; 1.0005x over previous
import functools

import jax
import jax.numpy as jnp
from jax import lax
from jax.experimental import pallas as pl
from jax.experimental.pallas import tpu as pltpu
from jax.experimental.pallas import tpu_sc as plsc

NC = 2
NS = 16
NW = NC * NS

CHUNK = 1600
NBUF = 2


def _gather_body(n_per_w, n_chunks, idx_hbm, table_hbm, out_hbm,
                 idx_v, rows_v, sem_i0, sem_i1, sem_g, sem_s0, sem_s1):
    wid = lax.axis_index("s") * NC + lax.axis_index("c")
    base = wid * n_per_w
    sem_i = (sem_i0, sem_i1)
    sem_s = (sem_s0, sem_s1)

    def start_idx(j, p):
        return pltpu.async_copy(
            idx_hbm.at[pl.ds(base + j * CHUNK, CHUNK)], idx_v.at[p], sem_i[p])

    idx_pending = {0: start_idx(0, 0), 1: start_idx(1, 1)}
    store_pending = {}

    for j in range(n_chunks):
        p = j % NBUF
        if j >= NBUF:
            store_pending.pop(j - NBUF).wait()
        idx_pending.pop(j).wait()
        gather = pltpu.async_copy(table_hbm.at[idx_v.at[p]], rows_v.at[p],
                                  sem_g)
        gather.wait()
        if j + NBUF < n_chunks:
            idx_pending[j + NBUF] = start_idx(j + NBUF, p)
        store_pending[j] = pltpu.async_copy(
            rows_v.at[p], out_hbm.at[pl.ds(base + j * CHUNK, CHUNK)], sem_s[p])

    for j in sorted(store_pending):
        store_pending.pop(j).wait()


def kernel(input, weight):
    B0, B1 = input.shape
    V, D = weight.shape
    N = B0 * B1
    assert N % (NW * CHUNK) == 0
    n_per_w = N // NW
    n_chunks = n_per_w // CHUNK

    idx = input.reshape(N).astype(jnp.int32)

    mesh = plsc.VectorSubcoreMesh(core_axis_name="c", subcore_axis_name="s")
    run = pl.kernel(
        functools.partial(_gather_body, n_per_w, n_chunks),
        out_type=jax.ShapeDtypeStruct((N, D), jnp.float32),
        mesh=mesh,
        scratch_types=[
            pltpu.VMEM((NBUF, CHUNK), jnp.int32),
            pltpu.VMEM((NBUF, CHUNK, D), jnp.float32),
            pltpu.SemaphoreType.DMA,
            pltpu.SemaphoreType.DMA,
            pltpu.SemaphoreType.DMA,
            pltpu.SemaphoreType.DMA,
            pltpu.SemaphoreType.DMA,
        ],
        compiler_params=pltpu.CompilerParams(use_tc_tiling_on_sc=False),
    )
    out = run(idx, weight)
    return out.reshape(B0, B1, D)
